# Initial kernel scaffold; baseline (speedup 1.0000x reference)
import jax
import jax.numpy as jnp
from jax.experimental import pallas as pl


def kernel(x, w1, b1, g1, be1, m1, v1, w2, b2, g2, be2, m2, v2):
    raise NotImplementedError("write your pallas kernel here")


# speedup vs baseline: 1.5379x; 1.5379x over previous
import jax
import jax.numpy as jnp
from jax.experimental import pallas as pl
from jax.experimental.pallas import tpu as pltpu

_H = 28
_W = 28
_WS = 32
_C = 128
_PIMG = (_H + 2) * _WS
_SLACK = 72


def _bb_kernel(x_ref, w1_ref, w2_ref, b1_ref, b2_ref, o_ref):
    Nb = x_ref.shape[0]
    G = Nb * _PIMG
    XT = G + _SLACK

    xf = x_ref[...].reshape(Nb, _H * _WS, _C)
    xf960 = jnp.pad(xf, ((0, 0), (_WS, _WS), (0, 0))).reshape(G, _C)
    xfp = jnp.pad(xf960, ((0, XT - G), (0, 0)))
    identity = xfp[33:33 + G, :]
    xp = xfp.astype(jnp.bfloat16)

    def conv(xpad, w_ref):
        rr = jnp.concatenate([xpad[0:XT - 2, :], xpad[1:XT - 1, :],
                              xpad[2:XT, :]], axis=1)
        ps = [jnp.dot(rr[32 * dy:32 * dy + G, :],
                      w_ref[3 * _C * dy:3 * _C * (dy + 1), :],
                      preferred_element_type=jnp.float32)
              for dy in range(3)]
        return (ps[0] + ps[1]) + ps[2]

    y1 = jnp.maximum(conv(xp, w1_ref) + b1_ref[...], 0.0)

    r960 = jax.lax.broadcasted_iota(jnp.int32, (1, _PIMG, 1), 1)
    valid = ((r960 & 31) < _W) & (r960 < _H * _WS)
    y1m = jnp.where(valid, y1.reshape(Nb, _PIMG, _C), 0.0)
    y1m = y1m.reshape(G, _C).astype(jnp.bfloat16)
    xp2 = jnp.pad(y1m, ((33, XT - G - 33), (0, 0)))

    y2 = conv(xp2, w2_ref) + b2_ref[...]
    out = jnp.maximum(y2 + identity, 0.0)
    out = out.reshape(Nb, _PIMG, _C)[:, :_H * _WS, :]
    o_ref[...] = out.reshape(Nb, _H, _WS, _C)


def _fold(w, b, g, be, m, v, eps=1e-5):
    scale = g / jnp.sqrt(v + eps)
    bias = scale * (b - m) + be
    w_hwio = jnp.transpose(w, (2, 3, 1, 0)) * scale[None, None, None, :]
    wr = w_hwio.reshape(9 * _C, _C).astype(jnp.bfloat16)
    return wr, bias.reshape(1, _C).astype(jnp.float32)


def kernel(x, w1, b1, g1, be1, m1, v1, w2, b2, g2, be2, m2, v2):
    N = x.shape[0]
    w1r, b1r = _fold(w1, b1, g1, be1, m1, v1)
    w2r, b2r = _fold(w2, b2, g2, be2, m2, v2)

    x_nhwc = jnp.transpose(x, (0, 2, 3, 1))
    x_p = jnp.pad(x_nhwc, ((0, 0), (0, 0), (1, 3), (0, 0)))

    Nb = 4
    out = pl.pallas_call(
        _bb_kernel,
        out_shape=jax.ShapeDtypeStruct((N, _H, _WS, _C), jnp.float32),
        grid=(N // Nb,),
        in_specs=[
            pl.BlockSpec((Nb, _H, _WS, _C), lambda n: (n, 0, 0, 0)),
            pl.BlockSpec((9 * _C, _C), lambda n: (0, 0)),
            pl.BlockSpec((9 * _C, _C), lambda n: (0, 0)),
            pl.BlockSpec((1, _C), lambda n: (0, 0)),
            pl.BlockSpec((1, _C), lambda n: (0, 0)),
        ],
        out_specs=pl.BlockSpec((Nb, _H, _WS, _C), lambda n: (n, 0, 0, 0)),
        compiler_params=pltpu.CompilerParams(
            dimension_semantics=("parallel",)),
    )(x_p, w1r, w2r, b1r, b2r)

    return jnp.transpose(out[:, :, :_W, :], (0, 3, 1, 2))


# speedup vs baseline: 1.8165x; 1.1812x over previous
import jax
import jax.numpy as jnp
from jax.experimental import pallas as pl
from jax.experimental.pallas import tpu as pltpu

_H = 28
_W = 28
_WS = 32
_C = 128
_S = _H * _W
_PIMG = (_H + 2) * _WS
_SLACK = 72


def _bb_kernel(x_ref, w1_ref, w2_ref, b1_ref, b2_ref, eye_ref, o_ref):
    Nb = x_ref.shape[0]
    G = Nb * _PIMG
    XT = G + _SLACK

    eye_f = eye_ref[...]
    eye_bf = eye_f.astype(jnp.bfloat16)

    x_cm = x_ref[...]
    xbf_cm = x_cm.astype(jnp.bfloat16)

    def t_in(i):
        return jax.lax.dot_general(
            xbf_cm[i], eye_bf, (((0,), (0,)), ((), ())),
            preferred_element_type=jnp.float32).astype(jnp.bfloat16)
    x_sm = jnp.stack([t_in(i) for i in range(Nb)])

    x4 = jnp.pad(x_sm.reshape(Nb, _H, _W, _C),
                 ((0, 0), (0, 0), (0, _WS - _W), (0, 0)))
    xf960 = jnp.pad(x4.reshape(Nb, _H * _WS, _C),
                    ((0, 0), (_WS, _WS), (0, 0))).reshape(G, _C)
    xp = jnp.pad(xf960, ((0, XT - G), (0, 0)))

    def conv(xpad, w_ref):
        rr = jnp.concatenate([xpad[0:XT - 2, :], xpad[1:XT - 1, :],
                              xpad[2:XT, :]], axis=1)
        ps = [jnp.dot(rr[32 * dy:32 * dy + G, :],
                      w_ref[3 * _C * dy:3 * _C * (dy + 1), :],
                      preferred_element_type=jnp.float32)
              for dy in range(3)]
        return (ps[0] + ps[1]) + ps[2]

    xp1 = jnp.pad(xp[:XT - 1, :], ((1, 0), (0, 0)))
    y1 = jnp.maximum(conv(xp1, w1_ref) + b1_ref[...], 0.0)

    r960 = jax.lax.broadcasted_iota(jnp.int32, (1, _PIMG, 1), 1)
    valid = ((r960 & 31) < _W) & (r960 < _H * _WS)
    y1m = jnp.where(valid, y1.reshape(Nb, _PIMG, _C), 0.0)
    y1m = y1m.reshape(G, _C).astype(jnp.bfloat16)
    xp2 = jnp.pad(y1m, ((33, XT - G - 33), (0, 0)))

    y2 = conv(xp2, w2_ref) + b2_ref[...]

    y2c = y2.reshape(Nb, _PIMG, _C)[:, :_H * _WS, :].reshape(Nb, _H, _WS, _C)
    y2c = y2c[:, :, :_W, :].reshape(Nb, _S, _C)

    def t_out(i):
        return jax.lax.dot_general(
            eye_f, y2c[i], (((1,), (1,)), ((), ())),
            preferred_element_type=jnp.float32)
    y2_cm = jnp.stack([t_out(i) for i in range(Nb)])

    o_ref[...] = jnp.maximum(y2_cm + x_cm, 0.0)


def _fold(w, b, g, be, m, v, eps=1e-5):
    scale = g / jnp.sqrt(v + eps)
    bias = scale * (b - m) + be
    w_hwio = jnp.transpose(w, (2, 3, 1, 0)) * scale[None, None, None, :]
    wr = w_hwio.reshape(9 * _C, _C).astype(jnp.bfloat16)
    return wr, bias.reshape(1, _C).astype(jnp.float32)


def kernel(x, w1, b1, g1, be1, m1, v1, w2, b2, g2, be2, m2, v2):
    N = x.shape[0]
    w1r, b1r = _fold(w1, b1, g1, be1, m1, v1)
    w2r, b2r = _fold(w2, b2, g2, be2, m2, v2)

    Nb = 4
    x_flat = x.reshape(N, _C, _S)
    eye = jnp.eye(_C, dtype=jnp.float32)
    out = pl.pallas_call(
        _bb_kernel,
        out_shape=jax.ShapeDtypeStruct((N, _C, _S), jnp.float32),
        grid=(N // Nb,),
        in_specs=[
            pl.BlockSpec((Nb, _C, _S), lambda n: (n, 0, 0)),
            pl.BlockSpec((9 * _C, _C), lambda n: (0, 0)),
            pl.BlockSpec((9 * _C, _C), lambda n: (0, 0)),
            pl.BlockSpec((1, _C), lambda n: (0, 0)),
            pl.BlockSpec((1, _C), lambda n: (0, 0)),
            pl.BlockSpec((_C, _C), lambda n: (0, 0)),
        ],
        out_specs=pl.BlockSpec((Nb, _C, _S), lambda n: (n, 0, 0)),
        compiler_params=pltpu.CompilerParams(
            dimension_semantics=("parallel",)),
    )(x_flat, w1r, w2r, b1r, b2r, eye)
    return out.reshape(N, _C, _H, _W)
